# baseline (speedup 1.0000x reference)
import functools

import jax
import jax.numpy as jnp
from jax import lax
from jax.experimental import pallas as pl
from jax.experimental.pallas import tpu as pltpu
from jax.experimental.pallas import tpu_sc as plsc

_BATCH = 16384
_NUM_CORES = 2
_NUM_SUBCORES = 16
_NUM_WORKERS = _NUM_CORES * _NUM_SUBCORES
_B_PER_W = _BATCH // _NUM_WORKERS

_mesh = plsc.VectorSubcoreMesh(
    core_axis_name="c", subcore_axis_name="s", num_cores=_NUM_CORES
)

_NCHUNK = _B_PER_W // 128
_CHUNK = 128


@functools.partial(
    pl.kernel,
    mesh=_mesh,
    out_type=jax.ShapeDtypeStruct((_BATCH,), jnp.float32),
    scratch_types=[
        pltpu.VMEM((_NCHUNK, _CHUNK), jnp.int32),
        pltpu.VMEM((_NCHUNK, _CHUNK), jnp.float32),
        pltpu.SemaphoreType.DMA((_NCHUNK,)),
    ],
)
def _gather_sc(idx_hbm, weight_hbm, out_hbm, idx_v, vals_v, sem):
    wid = lax.axis_index("s") * _NUM_CORES + lax.axis_index("c")
    base = wid * _B_PER_W
    loads = []
    for c in range(_NCHUNK):
        loads.append(
            pltpu.async_copy(
                idx_hbm.at[pl.ds(base + c * _CHUNK, _CHUNK)], idx_v.at[c], sem.at[c]
            )
        )
    gathers = []
    for c in range(_NCHUNK):
        loads[c].wait()
        gathers.append(
            pltpu.async_copy(weight_hbm.at[idx_v.at[c]], vals_v.at[c], sem.at[c])
        )
    stores = []
    for c in range(_NCHUNK):
        gathers[c].wait()
        stores.append(
            pltpu.async_copy(
                vals_v.at[c], out_hbm.at[pl.ds(base + c * _CHUNK, _CHUNK)], sem.at[c]
            )
        )
    for c in range(_NCHUNK):
        stores[c].wait()


@jax.jit
def kernel(idx, weight):
    return _gather_sc(idx.astype(jnp.int32), weight)


# speedup vs baseline: 1.0548x; 1.0548x over previous
import functools

import jax
import jax.numpy as jnp
from jax import lax
from jax.experimental import pallas as pl
from jax.experimental.pallas import tpu as pltpu
from jax.experimental.pallas import tpu_sc as plsc

_BATCH = 16384
_NUM_SUBCORES = 16
_B_PER_W = _BATCH // _NUM_SUBCORES

_mesh = plsc.VectorSubcoreMesh(core_axis_name="c", subcore_axis_name="s", num_cores=1)

_NCHUNK = _B_PER_W // 128
_CHUNK = 128


@functools.partial(
    pl.kernel,
    mesh=_mesh,
    out_type=jax.ShapeDtypeStruct((_BATCH,), jnp.float32),
    scratch_types=[
        pltpu.VMEM((_NCHUNK, _CHUNK), jnp.int32),
        pltpu.VMEM((_NCHUNK, _CHUNK), jnp.float32),
        pltpu.SemaphoreType.DMA,
        pltpu.SemaphoreType.DMA((_NCHUNK,)),
    ],
)
def _gather_sc(idx_hbm, weight_hbm, out_hbm, idx_v, vals_v, sem_i, sem):
    sid = lax.axis_index("s")
    base = sid * _B_PER_W
    pltpu.async_copy(idx_hbm.at[sid], idx_v, sem_i).wait()
    gathers = []
    for c in range(_NCHUNK):
        gathers.append(
            pltpu.async_copy(weight_hbm.at[idx_v.at[c]], vals_v.at[c], sem.at[c])
        )
    stores = []
    for c in range(_NCHUNK):
        gathers[c].wait()
        stores.append(
            pltpu.async_copy(
                vals_v.at[c], out_hbm.at[pl.ds(base + c * _CHUNK, _CHUNK)], sem.at[c]
            )
        )
    for c in range(_NCHUNK):
        stores[c].wait()


@jax.jit
def kernel(idx, weight):
    idx3 = idx.astype(jnp.int32).reshape(_NUM_SUBCORES, _NCHUNK, _CHUNK)
    return _gather_sc(idx3, weight)
